# baseline (speedup 1.0000x reference)
import functools
import jax
import jax.numpy as jnp
from jax import lax
from jax.experimental import pallas as pl
from jax.experimental.pallas import tpu as pltpu
from jax.experimental.pallas import tpu_sc as plsc

EX = 8
TOPK = 2
DM = 1024
FF = 2048
SL = 2048
NPAIR = SL * TOPK
MB = 256
NT = (NPAIR + EX * (MB - 1) + MB - 1) // MB
TBUF = NT * MB
NEOT = ((NT + 15) // 16) * 16

NC, NS, LN = 2, 16, 16
NW = NC * NS
CH = NPAIR // NW
TPW = SL // NW


def _iota16():
    return lax.iota(jnp.int32, 16)


def _gate_body(x_ref, wg_ref, e0_ref, e1_ref, p0_ref, p1_ref):
    x = x_ref[...]
    wg = wg_ref[...]
    g = lax.dot_general(x, wg, (((1,), (1,)), ((), ())),
                        preferred_element_type=jnp.float32)
    ii = lax.broadcasted_iota(jnp.int32, (SL, EX), 1)
    m0 = jnp.max(g, axis=1, keepdims=True)
    e0 = jnp.min(jnp.where(g == m0, ii, EX), axis=1, keepdims=True)
    g2 = jnp.where(ii == e0, -jnp.inf, g)
    m1 = jnp.max(g2, axis=1, keepdims=True)
    e1 = jnp.min(jnp.where(g2 == m1, ii, EX), axis=1, keepdims=True)
    b = jnp.exp(m1 - m0)
    denom = 1.0 + b
    e0_ref[...] = e0
    e1_ref[...] = e1
    p0_ref[...] = 1.0 / denom
    p1_ref[...] = b / denom


def _gating(x, wg):
    f = jax.ShapeDtypeStruct((SL, 1), jnp.float32)
    i = jax.ShapeDtypeStruct((SL, 1), jnp.int32)
    return pl.pallas_call(_gate_body, out_shape=[i, i, f, f])(x, wg)


def _counts_body(eids_hbm, counts_hbm, eid_v, cnt_v, row_v):
    wid = lax.axis_index("s") * NC + lax.axis_index("c")
    pltpu.sync_copy(eids_hbm.at[pl.ds(wid * CH, CH)], eid_v)
    ii = _iota16()
    cnt = jnp.zeros((16,), jnp.int32)
    for v in range(CH // 16):
        ev = eid_v[pl.ds(v * 16, 16)]
        for e in range(EX):
            c = jnp.sum((ev == e).astype(jnp.int32))
            cnt = cnt + jnp.where(ii == e, c, 0)
    cnt_v[...] = cnt
    pltpu.sync_copy(cnt_v, counts_hbm.at[wid])
    row_v[...] = cnt


def _counts(eids):
    call = functools.partial(
        pl.kernel,
        mesh=plsc.VectorSubcoreMesh(core_axis_name="c", subcore_axis_name="s"),
        out_type=jax.ShapeDtypeStruct((NW, 16), jnp.int32),
        scratch_types=[
            pltpu.VMEM((CH,), jnp.int32),
            pltpu.VMEM((16,), jnp.int32),
            pltpu.VMEM((16,), jnp.int32),
        ],
        compiler_params=pltpu.CompilerParams(needs_layout_passes=False),
    )
    return call(_counts_body)(eids)


def _route_body(eids_hbm, counts_hbm, x_hbm,
                pos_hbm, eot_hbm, xs_hbm,
                eid_v, call_v, cnt_v, gend_v, pos_v, pos2_v, tok2_v,
                eot_v, rows_v, sem):
    wid = lax.axis_index("s") * NC + lax.axis_index("c")
    ii = _iota16()
    pltpu.sync_copy(eids_hbm.at[pl.ds(wid * CH, CH)], eid_v)
    pltpu.sync_copy(counts_hbm, call_v)

    prior = jnp.zeros((16,), jnp.int32)
    total = jnp.zeros((16,), jnp.int32)
    for w in range(NW):
        row = call_v[w]
        wv = jnp.full((16,), w, jnp.int32)
        prior = prior + jnp.where(wv < wid, row, 0)
        total = total + row
    cpad = ((total + (MB - 1)) // MB) * MB
    gend = plsc.cumsum(cpad)
    gbase = gend - cpad
    start = gbase + prior
    cnt_v[...] = start
    gend_v[...] = gend

    @pl.when(wid == 0)
    def _():
        for vi in range(NEOT // 16):
            t_m = (ii + vi * 16) * MB
            acc = jnp.zeros((16,), jnp.int32)
            for e in range(EX):
                ge = jnp.max(jnp.where(ii == e, gend, 0))
                acc = acc + (t_m >= ge).astype(jnp.int32)
            eot_v[pl.ds(vi * 16, 16)] = jnp.minimum(acc, EX - 1)
        pltpu.sync_copy(eot_v, eot_hbm)

    for v in range(CH // 16):
        ev = eid_v[pl.ds(v * 16, 16)]
        base = plsc.load_gather(cnt_v, [ev])
        rank = jnp.zeros((16,), jnp.int32)
        hist = jnp.zeros((16,), jnp.int32)
        for e in range(EX):
            m = ev == e
            mi = m.astype(jnp.int32)
            cs = plsc.cumsum(mi)
            rank = rank + jnp.where(m, cs - 1, 0)
            hist = hist + jnp.where(ii == e, jnp.sum(mi), 0)
        posv = base + rank
        pos_v[pl.ds(v * 16, 16)] = posv
        pos2_v[v // 4, pl.ds((v % 4) * 16, 16)] = posv
        cnt_v[...] = cnt_v[...] + hist
    pltpu.sync_copy(pos_v, pos_hbm.at[pl.ds(wid * CH, CH)])

    tok_base = jnp.where(wid >= (SL // CH), wid * CH - SL, wid * CH)
    for v in range(CH // 16):
        tok2_v[v // 4, pl.ds((v % 4) * 16, 16)] = tok_base + v * 16 + ii
    for chk in range(CH // 64):
        pltpu.async_copy(x_hbm.at[tok2_v.at[chk]], rows_v, sem).wait()
        pltpu.async_copy(rows_v, xs_hbm.at[pos2_v.at[chk]], sem).wait()


def _route(eids, counts, x):
    call = functools.partial(
        pl.kernel,
        mesh=plsc.VectorSubcoreMesh(core_axis_name="c", subcore_axis_name="s"),
        out_type=[
            jax.ShapeDtypeStruct((NPAIR,), jnp.int32),
            jax.ShapeDtypeStruct((NEOT,), jnp.int32),
            jax.ShapeDtypeStruct((TBUF, DM), jnp.float32),
        ],
        scratch_types=[
            pltpu.VMEM((CH,), jnp.int32),
            pltpu.VMEM((NW, 16), jnp.int32),
            pltpu.VMEM((16,), jnp.int32),
            pltpu.VMEM((16,), jnp.int32),
            pltpu.VMEM((CH,), jnp.int32),
            pltpu.VMEM((CH // 64, 64), jnp.int32),
            pltpu.VMEM((CH // 64, 64), jnp.int32),
            pltpu.VMEM((NEOT,), jnp.int32),
            pltpu.VMEM((64, DM), jnp.float32),
            pltpu.SemaphoreType.DMA,
        ],
        compiler_params=pltpu.CompilerParams(needs_layout_passes=False),
    )
    return call(_route_body)(eids, counts, x)


def _mlp_body(eot_s, xs_ref, w1_ref, w2_ref, w3_ref, ys_ref):
    del eot_s
    xt = xs_ref[...].astype(jnp.bfloat16)
    w1 = w1_ref[0]
    w2 = w2_ref[0]
    w3 = w3_ref[0]
    h1 = lax.dot_general(xt, w1, (((1,), (1,)), ((), ())),
                         preferred_element_type=jnp.float32)
    h2 = lax.dot_general(xt, w2, (((1,), (1,)), ((), ())),
                         preferred_element_type=jnp.float32)
    h = (h1 * (1.0 / (1.0 + jnp.exp(-h1))) * h2).astype(jnp.bfloat16)
    ys_ref[...] = lax.dot_general(h, w3, (((1,), (1,)), ((), ())),
                                  preferred_element_type=jnp.float32)


def _mlp(eot, xs, w1, w2, w3):
    grid_spec = pltpu.PrefetchScalarGridSpec(
        num_scalar_prefetch=1,
        grid=(NT,),
        in_specs=[
            pl.BlockSpec((MB, DM), lambda t, eot_s: (t, 0)),
            pl.BlockSpec((1, FF, DM), lambda t, eot_s: (eot_s[t], 0, 0)),
            pl.BlockSpec((1, FF, DM), lambda t, eot_s: (eot_s[t], 0, 0)),
            pl.BlockSpec((1, DM, FF), lambda t, eot_s: (eot_s[t], 0, 0)),
        ],
        out_specs=pl.BlockSpec((MB, DM), lambda t, eot_s: (t, 0)),
    )
    return pl.pallas_call(
        _mlp_body,
        grid_spec=grid_spec,
        out_shape=jax.ShapeDtypeStruct((TBUF, DM), jnp.float32),
        compiler_params=pltpu.CompilerParams(
            dimension_semantics=("arbitrary",),
            vmem_limit_bytes=100 * 1024 * 1024),
    )(eot, xs, w1, w2, w3)


def _combine_body(ys_hbm, pos_hbm, prob_hbm, out_hbm,
                  idxa_v, idxb_v, pa_v, pb_v, rowsa_v, rowsb_v, outc_v, sem):
    wid = lax.axis_index("s") * NC + lax.axis_index("c")
    for chk in range(2):
        tb = wid * TPW + chk * (TPW // 2)
        n = TPW // 2
        pltpu.sync_copy(pos_hbm.at[pl.ds(tb, n)], idxa_v)
        pltpu.sync_copy(pos_hbm.at[pl.ds(SL + tb, n)], idxb_v)
        pltpu.sync_copy(prob_hbm.at[pl.ds(tb, n)], pa_v)
        pltpu.sync_copy(prob_hbm.at[pl.ds(SL + tb, n)], pb_v)
        pltpu.async_copy(ys_hbm.at[idxa_v], rowsa_v, sem).wait()
        pltpu.async_copy(ys_hbm.at[idxb_v], rowsb_v, sem).wait()

        def body(tt, carry):
            pa = plsc.load_gather(pa_v, [jnp.full((16,), tt, jnp.int32)])
            pb = plsc.load_gather(pb_v, [jnp.full((16,), tt, jnp.int32)])
            for d in range(DM // 16):
                sl = pl.ds(d * 16, 16)
                outc_v[tt, sl] = pa * rowsa_v[tt, sl] + pb * rowsb_v[tt, sl]
            return carry

        lax.fori_loop(0, n, body, 0)
        pltpu.sync_copy(outc_v, out_hbm.at[pl.ds(tb, n)])


def _combine(ys, pos, prob):
    call = functools.partial(
        pl.kernel,
        mesh=plsc.VectorSubcoreMesh(core_axis_name="c", subcore_axis_name="s"),
        out_type=jax.ShapeDtypeStruct((SL, DM), jnp.float32),
        scratch_types=[
            pltpu.VMEM((TPW // 2,), jnp.int32),
            pltpu.VMEM((TPW // 2,), jnp.int32),
            pltpu.VMEM((TPW // 2,), jnp.float32),
            pltpu.VMEM((TPW // 2,), jnp.float32),
            pltpu.VMEM((TPW // 2, DM), jnp.float32),
            pltpu.VMEM((TPW // 2, DM), jnp.float32),
            pltpu.VMEM((TPW // 2, DM), jnp.float32),
            pltpu.SemaphoreType.DMA,
        ],
        compiler_params=pltpu.CompilerParams(needs_layout_passes=False),
    )
    return call(_combine_body)(ys, pos, prob)


def kernel(xmat, Wg, W1, W2, W3):
    bsz, ln, _ = xmat.shape
    x = xmat.reshape(SL, DM)
    e0, e1, p0, p1 = _gating(x, Wg)
    eids = jnp.concatenate([e0[:, 0], e1[:, 0]])
    probs = jnp.concatenate([p0[:, 0], p1[:, 0]])
    counts = _counts(eids)
    pos, eot, xs = _route(eids, counts, x)
    ys = _mlp(eot, xs, W1.astype(jnp.bfloat16), W2.astype(jnp.bfloat16),
              W3.astype(jnp.bfloat16))
    out = _combine(ys, pos, probs)
    return out.reshape(bsz, ln, DM)


# speedup vs baseline: 1.3971x; 1.3971x over previous
import functools
import jax
import jax.numpy as jnp
from jax import lax
from jax.experimental import pallas as pl
from jax.experimental.pallas import tpu as pltpu
from jax.experimental.pallas import tpu_sc as plsc

EX = 8
TOPK = 2
DM = 1024
FF = 2048
SL = 2048
NPAIR = SL * TOPK
MB = 256
NT = (NPAIR + EX * (MB - 1) + MB - 1) // MB
TBUF = NT * MB
NEOT = ((NT + 15) // 16) * 16

NC, NS, LN = 2, 16, 16
NW = NC * NS
CH = NPAIR // NW
TPW = SL // NW


def _iota16():
    return lax.iota(jnp.int32, 16)


def _gate_body(x_ref, wg_ref, e0_ref, e1_ref, p0_ref, p1_ref):
    x = x_ref[...]
    wg = wg_ref[...]
    g = lax.dot_general(x, wg, (((1,), (1,)), ((), ())),
                        preferred_element_type=jnp.float32)
    ii = lax.broadcasted_iota(jnp.int32, (SL, EX), 1)
    m0 = jnp.max(g, axis=1, keepdims=True)
    e0 = jnp.min(jnp.where(g == m0, ii, EX), axis=1, keepdims=True)
    g2 = jnp.where(ii == e0, -jnp.inf, g)
    m1 = jnp.max(g2, axis=1, keepdims=True)
    e1 = jnp.min(jnp.where(g2 == m1, ii, EX), axis=1, keepdims=True)
    b = jnp.exp(m1 - m0)
    denom = 1.0 + b
    e0_ref[...] = e0
    e1_ref[...] = e1
    p0_ref[...] = 1.0 / denom
    p1_ref[...] = b / denom


def _gating(x, wg):
    f = jax.ShapeDtypeStruct((SL, 1), jnp.float32)
    i = jax.ShapeDtypeStruct((SL, 1), jnp.int32)
    return pl.pallas_call(_gate_body, out_shape=[i, i, f, f])(x, wg)


def _counts_body(eids_hbm, counts_hbm, eid_v, cnt_v, row_v):
    wid = lax.axis_index("s") * NC + lax.axis_index("c")
    pltpu.sync_copy(eids_hbm.at[pl.ds(wid * CH, CH)], eid_v)
    ii = _iota16()
    cnt = jnp.zeros((16,), jnp.int32)
    for v in range(CH // 16):
        ev = eid_v[pl.ds(v * 16, 16)]
        for e in range(EX):
            c = jnp.sum((ev == e).astype(jnp.int32))
            cnt = cnt + jnp.where(ii == e, c, 0)
    cnt_v[...] = cnt
    pltpu.sync_copy(cnt_v, counts_hbm.at[wid])
    row_v[...] = cnt


def _counts(eids):
    call = functools.partial(
        pl.kernel,
        mesh=plsc.VectorSubcoreMesh(core_axis_name="c", subcore_axis_name="s"),
        out_type=jax.ShapeDtypeStruct((NW, 16), jnp.int32),
        scratch_types=[
            pltpu.VMEM((CH,), jnp.int32),
            pltpu.VMEM((16,), jnp.int32),
            pltpu.VMEM((16,), jnp.int32),
        ],
        compiler_params=pltpu.CompilerParams(needs_layout_passes=False),
    )
    return call(_counts_body)(eids)


def _route_body(eids_hbm, counts_hbm, x_hbm,
                pos_hbm, eot_hbm, xs_hbm,
                eid_v, call_v, cnt_v, gend_v, pos_v, pos2_v, tok2_v,
                eot_v, rows_v, sem):
    wid = lax.axis_index("s") * NC + lax.axis_index("c")
    ii = _iota16()
    pltpu.sync_copy(eids_hbm.at[pl.ds(wid * CH, CH)], eid_v)
    pltpu.sync_copy(counts_hbm, call_v)

    prior = jnp.zeros((16,), jnp.int32)
    total = jnp.zeros((16,), jnp.int32)
    for w in range(NW):
        row = call_v[w]
        wv = jnp.full((16,), w, jnp.int32)
        prior = prior + jnp.where(wv < wid, row, 0)
        total = total + row
    cpad = ((total + (MB - 1)) // MB) * MB
    gend = plsc.cumsum(cpad)
    gbase = gend - cpad
    start = gbase + prior
    cnt_v[...] = start
    gend_v[...] = gend

    @pl.when(wid == 0)
    def _():
        ntv = jnp.max(jnp.where(ii == EX - 1, gend, 0)) // MB
        for vi in range(NEOT // 16):
            t_m = (ii + vi * 16) * MB
            acc = jnp.zeros((16,), jnp.int32)
            for e in range(EX):
                ge = jnp.max(jnp.where(ii == e, gend, 0))
                acc = acc + (t_m >= ge).astype(jnp.int32)
            eotv = jnp.minimum(acc, EX - 1)
            if vi * 16 <= NT < (vi + 1) * 16:
                eotv = jnp.where(ii == (NT - vi * 16), ntv, eotv)
            eot_v[pl.ds(vi * 16, 16)] = eotv
        pltpu.sync_copy(eot_v, eot_hbm)

    for v in range(CH // 16):
        ev = eid_v[pl.ds(v * 16, 16)]
        base = plsc.load_gather(cnt_v, [ev])
        rank = jnp.zeros((16,), jnp.int32)
        hist = jnp.zeros((16,), jnp.int32)
        for e in range(EX):
            m = ev == e
            mi = m.astype(jnp.int32)
            cs = plsc.cumsum(mi)
            rank = rank + jnp.where(m, cs - 1, 0)
            hist = hist + jnp.where(ii == e, jnp.sum(mi), 0)
        posv = base + rank
        pos_v[pl.ds(v * 16, 16)] = posv
        pos2_v[v // 4, pl.ds((v % 4) * 16, 16)] = posv
        cnt_v[...] = cnt_v[...] + hist
    pltpu.sync_copy(pos_v, pos_hbm.at[pl.ds(wid * CH, CH)])

    tok_base = jnp.where(wid >= (SL // CH), wid * CH - SL, wid * CH)
    for v in range(CH // 16):
        tok2_v[v // 4, pl.ds((v % 4) * 16, 16)] = tok_base + v * 16 + ii
    for chk in range(CH // 64):
        pltpu.async_copy(x_hbm.at[tok2_v.at[chk]], rows_v, sem).wait()
        pltpu.async_copy(rows_v, xs_hbm.at[pos2_v.at[chk]], sem).wait()


def _route(eids, counts, x):
    call = functools.partial(
        pl.kernel,
        mesh=plsc.VectorSubcoreMesh(core_axis_name="c", subcore_axis_name="s"),
        out_type=[
            jax.ShapeDtypeStruct((NPAIR,), jnp.int32),
            jax.ShapeDtypeStruct((NEOT,), jnp.int32),
            jax.ShapeDtypeStruct((TBUF, DM), jnp.float32),
        ],
        scratch_types=[
            pltpu.VMEM((CH,), jnp.int32),
            pltpu.VMEM((NW, 16), jnp.int32),
            pltpu.VMEM((16,), jnp.int32),
            pltpu.VMEM((16,), jnp.int32),
            pltpu.VMEM((CH,), jnp.int32),
            pltpu.VMEM((CH // 64, 64), jnp.int32),
            pltpu.VMEM((CH // 64, 64), jnp.int32),
            pltpu.VMEM((NEOT,), jnp.int32),
            pltpu.VMEM((64, DM), jnp.float32),
            pltpu.SemaphoreType.DMA,
        ],
        compiler_params=pltpu.CompilerParams(needs_layout_passes=False),
    )
    return call(_route_body)(eids, counts, x)


def _mlp_body(eot_s, xs_ref, w1_ref, w2_ref, w3_ref, ys_ref):
    t = pl.program_id(0)
    nt = eot_s[NT]

    @pl.when(t < nt)
    def _():
        _mlp_tile(xs_ref, w1_ref, w2_ref, w3_ref, ys_ref)


def _mlp_tile(xs_ref, w1_ref, w2_ref, w3_ref, ys_ref):
    xt = xs_ref[...]
    w1 = w1_ref[0]
    w2 = w2_ref[0]
    w3 = w3_ref[0]
    h1 = lax.dot_general(xt, w1, (((1,), (1,)), ((), ())),
                         preferred_element_type=jnp.float32)
    h2 = lax.dot_general(xt, w2, (((1,), (1,)), ((), ())),
                         preferred_element_type=jnp.float32)
    h = h1 * (1.0 / (1.0 + jnp.exp(-h1))) * h2
    ys_ref[...] = lax.dot_general(h, w3, (((1,), (1,)), ((), ())),
                                  preferred_element_type=jnp.float32)


def _mlp(eot, xs, w1, w2, w3):
    grid_spec = pltpu.PrefetchScalarGridSpec(
        num_scalar_prefetch=1,
        grid=(NT,),
        in_specs=[
            pl.BlockSpec(
                (MB, DM),
                lambda t, eot_s: (jnp.minimum(t, eot_s[NT] - 1), 0)),
            pl.BlockSpec(
                (1, FF, DM),
                lambda t, eot_s: (eot_s[jnp.minimum(t, eot_s[NT] - 1)], 0, 0)),
            pl.BlockSpec(
                (1, FF, DM),
                lambda t, eot_s: (eot_s[jnp.minimum(t, eot_s[NT] - 1)], 0, 0)),
            pl.BlockSpec(
                (1, DM, FF),
                lambda t, eot_s: (eot_s[jnp.minimum(t, eot_s[NT] - 1)], 0, 0)),
        ],
        out_specs=pl.BlockSpec(
            (MB, DM), lambda t, eot_s: (jnp.minimum(t, eot_s[NT] - 1), 0)),
    )
    return pl.pallas_call(
        _mlp_body,
        grid_spec=grid_spec,
        out_shape=jax.ShapeDtypeStruct((TBUF, DM), jnp.float32),
        compiler_params=pltpu.CompilerParams(
            dimension_semantics=("arbitrary",),
            vmem_limit_bytes=100 * 1024 * 1024),
    )(eot, xs, w1, w2, w3)


def _combine_body(ys_hbm, pos_hbm, prob_hbm, out_hbm,
                  idxa_v, idxb_v, pa_v, pb_v, rowsa_v, rowsb_v, outc_v, sem):
    wid = lax.axis_index("s") * NC + lax.axis_index("c")
    for chk in range(2):
        tb = wid * TPW + chk * (TPW // 2)
        n = TPW // 2
        pltpu.sync_copy(pos_hbm.at[pl.ds(tb, n)], idxa_v)
        pltpu.sync_copy(pos_hbm.at[pl.ds(SL + tb, n)], idxb_v)
        pltpu.sync_copy(prob_hbm.at[pl.ds(tb, n)], pa_v)
        pltpu.sync_copy(prob_hbm.at[pl.ds(SL + tb, n)], pb_v)
        pltpu.async_copy(ys_hbm.at[idxa_v], rowsa_v, sem).wait()
        pltpu.async_copy(ys_hbm.at[idxb_v], rowsb_v, sem).wait()

        def body(tt, carry):
            pa = plsc.load_gather(pa_v, [jnp.full((16,), tt, jnp.int32)])
            pb = plsc.load_gather(pb_v, [jnp.full((16,), tt, jnp.int32)])
            for d in range(DM // 16):
                sl = pl.ds(d * 16, 16)
                outc_v[tt, sl] = pa * rowsa_v[tt, sl] + pb * rowsb_v[tt, sl]
            return carry

        lax.fori_loop(0, n, body, 0)
        pltpu.sync_copy(outc_v, out_hbm.at[pl.ds(tb, n)])


def _combine(ys, pos, prob):
    call = functools.partial(
        pl.kernel,
        mesh=plsc.VectorSubcoreMesh(core_axis_name="c", subcore_axis_name="s"),
        out_type=jax.ShapeDtypeStruct((SL, DM), jnp.float32),
        scratch_types=[
            pltpu.VMEM((TPW // 2,), jnp.int32),
            pltpu.VMEM((TPW // 2,), jnp.int32),
            pltpu.VMEM((TPW // 2,), jnp.float32),
            pltpu.VMEM((TPW // 2,), jnp.float32),
            pltpu.VMEM((TPW // 2, DM), jnp.float32),
            pltpu.VMEM((TPW // 2, DM), jnp.float32),
            pltpu.VMEM((TPW // 2, DM), jnp.float32),
            pltpu.SemaphoreType.DMA,
        ],
        compiler_params=pltpu.CompilerParams(needs_layout_passes=False),
    )
    return call(_combine_body)(ys, pos, prob)


def kernel(xmat, Wg, W1, W2, W3):
    bsz, ln, _ = xmat.shape
    x = xmat.reshape(SL, DM)
    e0, e1, p0, p1 = _gating(x, Wg)
    eids = jnp.concatenate([e0[:, 0], e1[:, 0]])
    probs = jnp.concatenate([p0[:, 0], p1[:, 0]])
    counts = _counts(eids)
    pos, eot, xs = _route(eids, counts, x)
    ys = _mlp(eot, xs, W1, W2, W3)
    out = _combine(ys, pos, probs)
    return out.reshape(bsz, ln, DM)


# speedup vs baseline: 1.4088x; 1.0084x over previous
import functools
import jax
import jax.numpy as jnp
from jax import lax
from jax.experimental import pallas as pl
from jax.experimental.pallas import tpu as pltpu
from jax.experimental.pallas import tpu_sc as plsc

EX = 8
TOPK = 2
DM = 1024
FF = 2048
SL = 2048
NPAIR = SL * TOPK
MB = 256
NT = (NPAIR + EX * (MB - 1) + MB - 1) // MB
TBUF = NT * MB
NEOT = ((NT + 15) // 16) * 16

NC, NS, LN = 2, 16, 16
NW = NC * NS
CH = NPAIR // NW
TPW = SL // NW


def _iota16():
    return lax.iota(jnp.int32, 16)


def _gate_body(x_ref, wg_ref, e0_ref, e1_ref, p0_ref, p1_ref):
    x = x_ref[...]
    wg = wg_ref[...]
    g = lax.dot_general(x, wg, (((1,), (1,)), ((), ())),
                        preferred_element_type=jnp.float32)
    ii = lax.broadcasted_iota(jnp.int32, (SL, EX), 1)
    m0 = jnp.max(g, axis=1, keepdims=True)
    e0 = jnp.min(jnp.where(g == m0, ii, EX), axis=1, keepdims=True)
    g2 = jnp.where(ii == e0, -jnp.inf, g)
    m1 = jnp.max(g2, axis=1, keepdims=True)
    e1 = jnp.min(jnp.where(g2 == m1, ii, EX), axis=1, keepdims=True)
    b = jnp.exp(m1 - m0)
    denom = 1.0 + b
    e0_ref[...] = e0
    e1_ref[...] = e1
    p0_ref[...] = 1.0 / denom
    p1_ref[...] = b / denom


def _gating(x, wg):
    f = jax.ShapeDtypeStruct((SL, 1), jnp.float32)
    i = jax.ShapeDtypeStruct((SL, 1), jnp.int32)
    return pl.pallas_call(_gate_body, out_shape=[i, i, f, f])(x, wg)


def _counts_body(eids_hbm, counts_hbm, eid_v, cnt_v, row_v):
    wid = lax.axis_index("s") * NC + lax.axis_index("c")
    pltpu.sync_copy(eids_hbm.at[pl.ds(wid * CH, CH)], eid_v)
    ii = _iota16()
    cnt = jnp.zeros((16,), jnp.int32)
    for v in range(CH // 16):
        ev = eid_v[pl.ds(v * 16, 16)]
        for e in range(EX):
            c = jnp.sum((ev == e).astype(jnp.int32))
            cnt = cnt + jnp.where(ii == e, c, 0)
    cnt_v[...] = cnt
    pltpu.sync_copy(cnt_v, counts_hbm.at[wid])
    row_v[...] = cnt


def _counts(eids):
    call = functools.partial(
        pl.kernel,
        mesh=plsc.VectorSubcoreMesh(core_axis_name="c", subcore_axis_name="s"),
        out_type=jax.ShapeDtypeStruct((NW, 16), jnp.int32),
        scratch_types=[
            pltpu.VMEM((CH,), jnp.int32),
            pltpu.VMEM((16,), jnp.int32),
            pltpu.VMEM((16,), jnp.int32),
        ],
        compiler_params=pltpu.CompilerParams(needs_layout_passes=False),
    )
    return call(_counts_body)(eids)


def _route_body(eids_hbm, counts_hbm, x_hbm,
                pos_hbm, eot_hbm, xs_hbm,
                eid_v, call_v, cnt_v, gend_v, pos_v, pos2_v, tok2_v,
                eot_v, rows_v, sem, sem2, sem3, sem4):
    wid = lax.axis_index("s") * NC + lax.axis_index("c")
    ii = _iota16()
    pltpu.sync_copy(eids_hbm.at[pl.ds(wid * CH, CH)], eid_v)
    pltpu.sync_copy(counts_hbm, call_v)

    prior = jnp.zeros((16,), jnp.int32)
    total = jnp.zeros((16,), jnp.int32)
    for w in range(NW):
        row = call_v[w]
        wv = jnp.full((16,), w, jnp.int32)
        prior = prior + jnp.where(wv < wid, row, 0)
        total = total + row
    cpad = ((total + (MB - 1)) // MB) * MB
    gend = plsc.cumsum(cpad)
    gbase = gend - cpad
    start = gbase + prior
    cnt_v[...] = start
    gend_v[...] = gend

    @pl.when(wid == 0)
    def _():
        ntv = jnp.max(jnp.where(ii == EX - 1, gend, 0)) // MB
        for vi in range(NEOT // 16):
            t_m = (ii + vi * 16) * MB
            acc = jnp.zeros((16,), jnp.int32)
            for e in range(EX):
                ge = jnp.max(jnp.where(ii == e, gend, 0))
                acc = acc + (t_m >= ge).astype(jnp.int32)
            eotv = jnp.minimum(acc, EX - 1)
            if vi * 16 <= NT < (vi + 1) * 16:
                eotv = jnp.where(ii == (NT - vi * 16), ntv, eotv)
            eot_v[pl.ds(vi * 16, 16)] = eotv
        pltpu.sync_copy(eot_v, eot_hbm)

    for v in range(CH // 16):
        ev = eid_v[pl.ds(v * 16, 16)]
        base = plsc.load_gather(cnt_v, [ev])
        rank = jnp.zeros((16,), jnp.int32)
        hist = jnp.zeros((16,), jnp.int32)
        for e in range(EX):
            m = ev == e
            mi = m.astype(jnp.int32)
            cs = plsc.cumsum(mi)
            rank = rank + jnp.where(m, cs - 1, 0)
            hist = hist + jnp.where(ii == e, jnp.sum(mi), 0)
        posv = base + rank
        pos_v[pl.ds(v * 16, 16)] = posv
        pos2_v[v // 2, pl.ds((v % 2) * 16, 16)] = posv
        cnt_v[...] = cnt_v[...] + hist
    pltpu.sync_copy(pos_v, pos_hbm.at[pl.ds(wid * CH, CH)])

    tok_base = jnp.where(wid >= (SL // CH), wid * CH - SL, wid * CH)
    for v in range(CH // 16):
        tok2_v[v // 2, pl.ds((v % 2) * 16, 16)] = tok_base + v * 16 + ii
    nchk = CH // 32
    rows = [rows_v.at[0], rows_v.at[1]]
    gsem = [sem, sem2]
    ssem = [sem3, sem4]
    pltpu.async_copy(x_hbm.at[tok2_v.at[0]], rows[0], gsem[0])
    pltpu.async_copy(x_hbm.at[tok2_v.at[1]], rows[1], gsem[1])
    for chk in range(nchk):
        b = chk % 2
        pltpu.make_async_copy(x_hbm.at[tok2_v.at[chk]], rows[b],
                              gsem[b]).wait()
        pltpu.async_copy(rows[b], xs_hbm.at[pos2_v.at[chk]], ssem[b])
        if chk + 2 < nchk:
            pltpu.make_async_copy(rows[b], xs_hbm.at[pos2_v.at[chk]],
                                  ssem[b]).wait()
            pltpu.async_copy(x_hbm.at[tok2_v.at[chk + 2]], rows[b], gsem[b])
    for chk in range(max(nchk - 2, 0), nchk):
        b = chk % 2
        pltpu.make_async_copy(rows[b], xs_hbm.at[pos2_v.at[chk]],
                              ssem[b]).wait()


def _route(eids, counts, x):
    call = functools.partial(
        pl.kernel,
        mesh=plsc.VectorSubcoreMesh(core_axis_name="c", subcore_axis_name="s"),
        out_type=[
            jax.ShapeDtypeStruct((NPAIR,), jnp.int32),
            jax.ShapeDtypeStruct((NEOT,), jnp.int32),
            jax.ShapeDtypeStruct((TBUF, DM), jnp.float32),
        ],
        scratch_types=[
            pltpu.VMEM((CH,), jnp.int32),
            pltpu.VMEM((NW, 16), jnp.int32),
            pltpu.VMEM((16,), jnp.int32),
            pltpu.VMEM((16,), jnp.int32),
            pltpu.VMEM((CH,), jnp.int32),
            pltpu.VMEM((CH // 32, 32), jnp.int32),
            pltpu.VMEM((CH // 32, 32), jnp.int32),
            pltpu.VMEM((NEOT,), jnp.int32),
            pltpu.VMEM((2, 32, DM), jnp.float32),
            pltpu.SemaphoreType.DMA,
            pltpu.SemaphoreType.DMA,
            pltpu.SemaphoreType.DMA,
            pltpu.SemaphoreType.DMA,
        ],
        compiler_params=pltpu.CompilerParams(needs_layout_passes=False),
    )
    return call(_route_body)(eids, counts, x)


def _mlp_body(eot_s, xs_ref, w1_ref, w2_ref, w3_ref, ys_ref):
    t = pl.program_id(0)
    nt = eot_s[NT]

    @pl.when(t < nt)
    def _():
        _mlp_tile(xs_ref, w1_ref, w2_ref, w3_ref, ys_ref)


def _mlp_tile(xs_ref, w1_ref, w2_ref, w3_ref, ys_ref):
    xt = xs_ref[...]
    w1 = w1_ref[0]
    w2 = w2_ref[0]
    w3 = w3_ref[0]
    h1 = lax.dot_general(xt, w1, (((1,), (1,)), ((), ())),
                         preferred_element_type=jnp.float32)
    h2 = lax.dot_general(xt, w2, (((1,), (1,)), ((), ())),
                         preferred_element_type=jnp.float32)
    h = h1 * (1.0 / (1.0 + jnp.exp(-h1))) * h2
    ys_ref[...] = lax.dot_general(h, w3, (((1,), (1,)), ((), ())),
                                  preferred_element_type=jnp.float32)


def _mlp(eot, xs, w1, w2, w3):
    grid_spec = pltpu.PrefetchScalarGridSpec(
        num_scalar_prefetch=1,
        grid=(NT,),
        in_specs=[
            pl.BlockSpec(
                (MB, DM),
                lambda t, eot_s: (jnp.minimum(t, eot_s[NT] - 1), 0)),
            pl.BlockSpec(
                (1, FF, DM),
                lambda t, eot_s: (eot_s[jnp.minimum(t, eot_s[NT] - 1)], 0, 0)),
            pl.BlockSpec(
                (1, FF, DM),
                lambda t, eot_s: (eot_s[jnp.minimum(t, eot_s[NT] - 1)], 0, 0)),
            pl.BlockSpec(
                (1, DM, FF),
                lambda t, eot_s: (eot_s[jnp.minimum(t, eot_s[NT] - 1)], 0, 0)),
        ],
        out_specs=pl.BlockSpec(
            (MB, DM), lambda t, eot_s: (jnp.minimum(t, eot_s[NT] - 1), 0)),
    )
    return pl.pallas_call(
        _mlp_body,
        grid_spec=grid_spec,
        out_shape=jax.ShapeDtypeStruct((TBUF, DM), jnp.float32),
        compiler_params=pltpu.CompilerParams(
            dimension_semantics=("arbitrary",),
            vmem_limit_bytes=100 * 1024 * 1024),
    )(eot, xs, w1, w2, w3)


def _combine_body(ys_hbm, pos_hbm, prob_hbm, out_hbm,
                  idxa_v, idxb_v, pa_v, pb_v, rowsa_v, rowsb_v, outc_v,
                  sem, semb):
    wid = lax.axis_index("s") * NC + lax.axis_index("c")
    for chk in range(2):
        tb = wid * TPW + chk * (TPW // 2)
        n = TPW // 2
        pltpu.sync_copy(pos_hbm.at[pl.ds(tb, n)], idxa_v)
        pltpu.sync_copy(pos_hbm.at[pl.ds(SL + tb, n)], idxb_v)
        pltpu.sync_copy(prob_hbm.at[pl.ds(tb, n)], pa_v)
        pltpu.sync_copy(prob_hbm.at[pl.ds(SL + tb, n)], pb_v)
        ca = pltpu.async_copy(ys_hbm.at[idxa_v], rowsa_v, sem)
        cb = pltpu.async_copy(ys_hbm.at[idxb_v], rowsb_v, semb)
        ca.wait()
        cb.wait()

        def body(tt, carry):
            pa = plsc.load_gather(pa_v, [jnp.full((16,), tt, jnp.int32)])
            pb = plsc.load_gather(pb_v, [jnp.full((16,), tt, jnp.int32)])
            for d in range(DM // 16):
                sl = pl.ds(d * 16, 16)
                outc_v[tt, sl] = pa * rowsa_v[tt, sl] + pb * rowsb_v[tt, sl]
            return carry

        lax.fori_loop(0, n, body, 0)
        pltpu.sync_copy(outc_v, out_hbm.at[pl.ds(tb, n)])


def _combine(ys, pos, prob):
    call = functools.partial(
        pl.kernel,
        mesh=plsc.VectorSubcoreMesh(core_axis_name="c", subcore_axis_name="s"),
        out_type=jax.ShapeDtypeStruct((SL, DM), jnp.float32),
        scratch_types=[
            pltpu.VMEM((TPW // 2,), jnp.int32),
            pltpu.VMEM((TPW // 2,), jnp.int32),
            pltpu.VMEM((TPW // 2,), jnp.float32),
            pltpu.VMEM((TPW // 2,), jnp.float32),
            pltpu.VMEM((TPW // 2, DM), jnp.float32),
            pltpu.VMEM((TPW // 2, DM), jnp.float32),
            pltpu.VMEM((TPW // 2, DM), jnp.float32),
            pltpu.SemaphoreType.DMA,
            pltpu.SemaphoreType.DMA,
        ],
        compiler_params=pltpu.CompilerParams(needs_layout_passes=False),
    )
    return call(_combine_body)(ys, pos, prob)


def kernel(xmat, Wg, W1, W2, W3):
    bsz, ln, _ = xmat.shape
    x = xmat.reshape(SL, DM)
    e0, e1, p0, p1 = _gating(x, Wg)
    eids = jnp.concatenate([e0[:, 0], e1[:, 0]])
    probs = jnp.concatenate([p0[:, 0], p1[:, 0]])
    counts = _counts(eids)
    pos, eot, xs = _route(eids, counts, x)
    ys = _mlp(eot, xs, W1, W2, W3)
    out = _combine(ys, pos, probs)
    return out.reshape(bsz, ln, DM)


# speedup vs baseline: 1.4520x; 1.0306x over previous
import functools
import jax
import jax.numpy as jnp
from jax import lax
from jax.experimental import pallas as pl
from jax.experimental.pallas import tpu as pltpu
from jax.experimental.pallas import tpu_sc as plsc

EX = 8
TOPK = 2
DM = 1024
FF = 2048
SL = 2048
NPAIR = SL * TOPK
MB = 256
NT = (NPAIR + EX * (MB - 1) + MB - 1) // MB
TBUF = NT * MB
NEOT = ((NT + 15) // 16) * 16

NC, NS, LN = 2, 16, 16
NW = NC * NS
CH = NPAIR // NW
TPW = SL // NW


def _iota16():
    return lax.iota(jnp.int32, 16)


def _gate_body(x_ref, wg_ref, eids_ref, probs_ref, counts_ref):
    x = x_ref[...]
    wg = wg_ref[...]
    g = lax.dot_general(x, wg, (((1,), (1,)), ((), ())),
                        preferred_element_type=jnp.float32)
    ii = lax.broadcasted_iota(jnp.int32, (SL, EX), 1)
    m0 = jnp.max(g, axis=1, keepdims=True)
    e0 = jnp.min(jnp.where(g == m0, ii, EX), axis=1, keepdims=True)
    g2 = jnp.where(ii == e0, -jnp.inf, g)
    m1 = jnp.max(g2, axis=1, keepdims=True)
    e1 = jnp.min(jnp.where(g2 == m1, ii, EX), axis=1, keepdims=True)
    b = jnp.exp(m1 - m0)
    denom = 1.0 + b
    eids_ref[pl.ds(0, SL), :] = e0
    eids_ref[pl.ds(SL, SL), :] = e1
    probs_ref[pl.ds(0, SL), :] = 1.0 / denom
    probs_ref[pl.ds(SL, SL), :] = b / denom
    i16 = lax.broadcasted_iota(jnp.int32, (SL, 16), 1)
    oh0 = (i16 == e0).astype(jnp.float32)
    oh1 = (i16 == e1).astype(jnp.float32)
    seg = lax.broadcasted_iota(jnp.int32, (SL // CH, SL), 0)
    col = lax.broadcasted_iota(jnp.int32, (SL // CH, SL), 1)
    sel = (col // CH == seg).astype(jnp.float32)
    c0 = lax.dot_general(sel, oh0, (((1,), (0,)), ((), ())),
                         preferred_element_type=jnp.float32)
    c1 = lax.dot_general(sel, oh1, (((1,), (0,)), ((), ())),
                         preferred_element_type=jnp.float32)
    counts_ref[pl.ds(0, SL // CH), :] = c0.astype(jnp.int32)
    counts_ref[pl.ds(SL // CH, SL // CH), :] = c1.astype(jnp.int32)


def _gating(x, wg):
    outs = [
        jax.ShapeDtypeStruct((NPAIR, 1), jnp.int32),
        jax.ShapeDtypeStruct((NPAIR, 1), jnp.float32),
        jax.ShapeDtypeStruct((NW, 16), jnp.int32),
    ]
    return pl.pallas_call(_gate_body, out_shape=outs)(x, wg)


def _route_body(eids_hbm, counts_hbm, x_hbm,
                pos_hbm, eot_hbm, xs_hbm,
                eid_v, call_v, cnt_v, gend_v, pos_v, pos2_v, tok2_v,
                eot_v, rows_v, sem, sem2, sem3, sem4):
    wid = lax.axis_index("s") * NC + lax.axis_index("c")
    ii = _iota16()
    pltpu.sync_copy(eids_hbm.at[pl.ds(wid * CH, CH)], eid_v)
    pltpu.sync_copy(counts_hbm, call_v)

    prior = jnp.zeros((16,), jnp.int32)
    total = jnp.zeros((16,), jnp.int32)
    for w in range(NW):
        row = call_v[w]
        wv = jnp.full((16,), w, jnp.int32)
        prior = prior + jnp.where(wv < wid, row, 0)
        total = total + row
    cpad = ((total + (MB - 1)) // MB) * MB
    gend = plsc.cumsum(cpad)
    gbase = gend - cpad
    start = gbase + prior
    cnt_v[...] = start
    gend_v[...] = gend

    @pl.when(wid == 0)
    def _():
        ntv = jnp.max(jnp.where(ii == EX - 1, gend, 0)) // MB
        for vi in range(NEOT // 16):
            t_m = (ii + vi * 16) * MB
            acc = jnp.zeros((16,), jnp.int32)
            for e in range(EX):
                ge = jnp.max(jnp.where(ii == e, gend, 0))
                acc = acc + (t_m >= ge).astype(jnp.int32)
            eotv = jnp.minimum(acc, EX - 1)
            if vi * 16 <= NT < (vi + 1) * 16:
                eotv = jnp.where(ii == (NT - vi * 16), ntv, eotv)
            eot_v[pl.ds(vi * 16, 16)] = eotv
        pltpu.sync_copy(eot_v, eot_hbm)

    for v in range(CH // 16):
        ev = eid_v[pl.ds(v * 16, 16)]
        base = plsc.load_gather(cnt_v, [ev])
        rank = jnp.zeros((16,), jnp.int32)
        hist = jnp.zeros((16,), jnp.int32)
        for e in range(EX):
            m = ev == e
            mi = m.astype(jnp.int32)
            cs = plsc.cumsum(mi)
            rank = rank + jnp.where(m, cs - 1, 0)
            hist = hist + jnp.where(ii == e, jnp.sum(mi), 0)
        posv = base + rank
        pos_v[pl.ds(v * 16, 16)] = posv
        pos2_v[v // 2, pl.ds((v % 2) * 16, 16)] = posv
        cnt_v[...] = cnt_v[...] + hist
    pltpu.sync_copy(pos_v, pos_hbm.at[pl.ds(wid * CH, CH)])

    tok_base = jnp.where(wid >= (SL // CH), wid * CH - SL, wid * CH)
    for v in range(CH // 16):
        tok2_v[v // 2, pl.ds((v % 2) * 16, 16)] = tok_base + v * 16 + ii
    nchk = CH // 32
    rows = [rows_v.at[0], rows_v.at[1]]
    gsem = [sem, sem2]
    ssem = [sem3, sem4]
    pltpu.async_copy(x_hbm.at[tok2_v.at[0]], rows[0], gsem[0])
    pltpu.async_copy(x_hbm.at[tok2_v.at[1]], rows[1], gsem[1])
    for chk in range(nchk):
        b = chk % 2
        pltpu.make_async_copy(x_hbm.at[tok2_v.at[chk]], rows[b],
                              gsem[b]).wait()
        pltpu.async_copy(rows[b], xs_hbm.at[pos2_v.at[chk]], ssem[b])
        if chk + 2 < nchk:
            pltpu.make_async_copy(rows[b], xs_hbm.at[pos2_v.at[chk]],
                                  ssem[b]).wait()
            pltpu.async_copy(x_hbm.at[tok2_v.at[chk + 2]], rows[b], gsem[b])
    for chk in range(max(nchk - 2, 0), nchk):
        b = chk % 2
        pltpu.make_async_copy(rows[b], xs_hbm.at[pos2_v.at[chk]],
                              ssem[b]).wait()


def _route(eids, counts, x):
    call = functools.partial(
        pl.kernel,
        mesh=plsc.VectorSubcoreMesh(core_axis_name="c", subcore_axis_name="s"),
        out_type=[
            jax.ShapeDtypeStruct((NPAIR,), jnp.int32),
            jax.ShapeDtypeStruct((NEOT,), jnp.int32),
            jax.ShapeDtypeStruct((TBUF, DM), jnp.float32),
        ],
        scratch_types=[
            pltpu.VMEM((CH,), jnp.int32),
            pltpu.VMEM((NW, 16), jnp.int32),
            pltpu.VMEM((16,), jnp.int32),
            pltpu.VMEM((16,), jnp.int32),
            pltpu.VMEM((CH,), jnp.int32),
            pltpu.VMEM((CH // 32, 32), jnp.int32),
            pltpu.VMEM((CH // 32, 32), jnp.int32),
            pltpu.VMEM((NEOT,), jnp.int32),
            pltpu.VMEM((2, 32, DM), jnp.float32),
            pltpu.SemaphoreType.DMA,
            pltpu.SemaphoreType.DMA,
            pltpu.SemaphoreType.DMA,
            pltpu.SemaphoreType.DMA,
        ],
        compiler_params=pltpu.CompilerParams(needs_layout_passes=False),
    )
    return call(_route_body)(eids, counts, x)


def _mlp_body(eot_s, xs_ref, w1_ref, w2_ref, w3_ref, ys_ref):
    t = pl.program_id(0)
    nt = eot_s[NT]

    @pl.when(t < nt)
    def _():
        _mlp_tile(xs_ref, w1_ref, w2_ref, w3_ref, ys_ref)


def _mlp_tile(xs_ref, w1_ref, w2_ref, w3_ref, ys_ref):
    xt = xs_ref[...]
    w1 = w1_ref[0]
    w2 = w2_ref[0]
    w3 = w3_ref[0]
    h1 = lax.dot_general(xt, w1, (((1,), (1,)), ((), ())),
                         preferred_element_type=jnp.float32)
    h2 = lax.dot_general(xt, w2, (((1,), (1,)), ((), ())),
                         preferred_element_type=jnp.float32)
    h = h1 * (1.0 / (1.0 + jnp.exp(-h1))) * h2
    ys_ref[...] = lax.dot_general(h, w3, (((1,), (1,)), ((), ())),
                                  preferred_element_type=jnp.float32)


def _mlp(eot, xs, w1, w2, w3):
    grid_spec = pltpu.PrefetchScalarGridSpec(
        num_scalar_prefetch=1,
        grid=(NT,),
        in_specs=[
            pl.BlockSpec(
                (MB, DM),
                lambda t, eot_s: (jnp.minimum(t, eot_s[NT] - 1), 0)),
            pl.BlockSpec(
                (1, FF, DM),
                lambda t, eot_s: (eot_s[jnp.minimum(t, eot_s[NT] - 1)], 0, 0)),
            pl.BlockSpec(
                (1, FF, DM),
                lambda t, eot_s: (eot_s[jnp.minimum(t, eot_s[NT] - 1)], 0, 0)),
            pl.BlockSpec(
                (1, DM, FF),
                lambda t, eot_s: (eot_s[jnp.minimum(t, eot_s[NT] - 1)], 0, 0)),
        ],
        out_specs=pl.BlockSpec(
            (MB, DM), lambda t, eot_s: (jnp.minimum(t, eot_s[NT] - 1), 0)),
    )
    return pl.pallas_call(
        _mlp_body,
        grid_spec=grid_spec,
        out_shape=jax.ShapeDtypeStruct((TBUF, DM), jnp.float32),
        compiler_params=pltpu.CompilerParams(
            dimension_semantics=("arbitrary",),
            vmem_limit_bytes=100 * 1024 * 1024),
    )(eot, xs, w1, w2, w3)


def _combine_body(ys_hbm, pos_hbm, prob_hbm, out_hbm,
                  idxa_v, idxb_v, pa_v, pb_v, rowsa_v, rowsb_v, outc_v,
                  sem, semb):
    wid = lax.axis_index("s") * NC + lax.axis_index("c")
    for chk in range(2):
        tb = wid * TPW + chk * (TPW // 2)
        n = TPW // 2
        pltpu.sync_copy(pos_hbm.at[pl.ds(tb, n)], idxa_v)
        pltpu.sync_copy(pos_hbm.at[pl.ds(SL + tb, n)], idxb_v)
        pltpu.sync_copy(prob_hbm.at[pl.ds(tb, n)], pa_v)
        pltpu.sync_copy(prob_hbm.at[pl.ds(SL + tb, n)], pb_v)
        ca = pltpu.async_copy(ys_hbm.at[idxa_v], rowsa_v, sem)
        cb = pltpu.async_copy(ys_hbm.at[idxb_v], rowsb_v, semb)
        ca.wait()
        cb.wait()

        def body(tt, carry):
            pa = plsc.load_gather(pa_v, [jnp.full((16,), tt, jnp.int32)])
            pb = plsc.load_gather(pb_v, [jnp.full((16,), tt, jnp.int32)])
            for d in range(DM // 16):
                sl = pl.ds(d * 16, 16)
                outc_v[tt, sl] = pa * rowsa_v[tt, sl] + pb * rowsb_v[tt, sl]
            return carry

        lax.fori_loop(0, n, body, 0)
        pltpu.sync_copy(outc_v, out_hbm.at[pl.ds(tb, n)])


def _combine(ys, pos, prob):
    call = functools.partial(
        pl.kernel,
        mesh=plsc.VectorSubcoreMesh(core_axis_name="c", subcore_axis_name="s"),
        out_type=jax.ShapeDtypeStruct((SL, DM), jnp.float32),
        scratch_types=[
            pltpu.VMEM((TPW // 2,), jnp.int32),
            pltpu.VMEM((TPW // 2,), jnp.int32),
            pltpu.VMEM((TPW // 2,), jnp.float32),
            pltpu.VMEM((TPW // 2,), jnp.float32),
            pltpu.VMEM((TPW // 2, DM), jnp.float32),
            pltpu.VMEM((TPW // 2, DM), jnp.float32),
            pltpu.VMEM((TPW // 2, DM), jnp.float32),
            pltpu.SemaphoreType.DMA,
            pltpu.SemaphoreType.DMA,
        ],
        compiler_params=pltpu.CompilerParams(needs_layout_passes=False),
    )
    return call(_combine_body)(ys, pos, prob)


def kernel(xmat, Wg, W1, W2, W3):
    bsz, ln, _ = xmat.shape
    x = xmat.reshape(SL, DM)
    eo, po, counts = _gating(x, Wg)
    eids = eo.reshape(NPAIR)
    probs = po.reshape(NPAIR)
    pos, eot, xs = _route(eids, counts, x)
    ys = _mlp(eot, xs, W1, W2, W3)
    out = _combine(ys, pos, probs)
    return out.reshape(bsz, ln, DM)


# speedup vs baseline: 1.4833x; 1.0216x over previous
import functools
import jax
import jax.numpy as jnp
from jax import lax
from jax.experimental import pallas as pl
from jax.experimental.pallas import tpu as pltpu
from jax.experimental.pallas import tpu_sc as plsc

EX = 8
TOPK = 2
DM = 1024
FF = 2048
SL = 2048
NPAIR = SL * TOPK
MB = 256
NT = (NPAIR + EX * (MB - 1) + MB - 1) // MB
TBUF = NT * MB
NEOT = ((NT + 15) // 16) * 16

NC, NS, LN = 2, 16, 16
NW = NC * NS
CH = NPAIR // NW
TPW = SL // NW


def _iota16():
    return lax.iota(jnp.int32, 16)


def _gate_body(x_ref, wg_ref, eids_ref, probs_ref, counts_ref):
    x = x_ref[...]
    wg = wg_ref[...]
    g = lax.dot_general(x, wg, (((1,), (1,)), ((), ())),
                        preferred_element_type=jnp.float32)
    ii = lax.broadcasted_iota(jnp.int32, (SL, EX), 1)
    m0 = jnp.max(g, axis=1, keepdims=True)
    e0 = jnp.min(jnp.where(g == m0, ii, EX), axis=1, keepdims=True)
    g2 = jnp.where(ii == e0, -jnp.inf, g)
    m1 = jnp.max(g2, axis=1, keepdims=True)
    e1 = jnp.min(jnp.where(g2 == m1, ii, EX), axis=1, keepdims=True)
    b = jnp.exp(m1 - m0)
    denom = 1.0 + b
    eids_ref[pl.ds(0, SL), :] = e0
    eids_ref[pl.ds(SL, SL), :] = e1
    probs_ref[pl.ds(0, SL), :] = 1.0 / denom
    probs_ref[pl.ds(SL, SL), :] = b / denom
    i16 = lax.broadcasted_iota(jnp.int32, (SL, 16), 1)
    oh0 = (i16 == e0).astype(jnp.float32)
    oh1 = (i16 == e1).astype(jnp.float32)
    seg = lax.broadcasted_iota(jnp.int32, (SL // CH, SL), 0)
    col = lax.broadcasted_iota(jnp.int32, (SL // CH, SL), 1)
    sel = (col // CH == seg).astype(jnp.float32)
    c0 = lax.dot_general(sel, oh0, (((1,), (0,)), ((), ())),
                         preferred_element_type=jnp.float32)
    c1 = lax.dot_general(sel, oh1, (((1,), (0,)), ((), ())),
                         preferred_element_type=jnp.float32)
    counts_ref[pl.ds(0, SL // CH), :] = c0.astype(jnp.int32)
    counts_ref[pl.ds(SL // CH, SL // CH), :] = c1.astype(jnp.int32)


def _gating(x, wg):
    outs = [
        jax.ShapeDtypeStruct((NPAIR, 1), jnp.int32),
        jax.ShapeDtypeStruct((NPAIR, 1), jnp.float32),
        jax.ShapeDtypeStruct((NW, 16), jnp.int32),
    ]
    return pl.pallas_call(_gate_body, out_shape=outs)(x, wg)


def _route_body(eids_hbm, counts_hbm, x_hbm,
                pos_hbm, eot_hbm, xs_hbm,
                eid_v, call_v, cnt_v, gend_v, pos_v, pos2_v, tok2_v,
                eot_v, rows_v, sem, sem2, sem3, sem4):
    wid = lax.axis_index("s") * NC + lax.axis_index("c")
    ii = _iota16()
    pltpu.sync_copy(eids_hbm.at[pl.ds(wid * CH, CH)], eid_v)
    pltpu.sync_copy(counts_hbm, call_v)

    prior = jnp.zeros((16,), jnp.int32)
    total = jnp.zeros((16,), jnp.int32)
    for w in range(NW):
        row = call_v[w]
        wv = jnp.full((16,), w, jnp.int32)
        prior = prior + jnp.where(wv < wid, row, 0)
        total = total + row
    cpad = ((total + (MB - 1)) // MB) * MB
    gend = plsc.cumsum(cpad)
    gbase = gend - cpad
    start = gbase + prior
    cnt_v[...] = start
    gend_v[...] = gend

    @pl.when(wid == 0)
    def _():
        ntv = jnp.max(jnp.where(ii == EX - 1, gend, 0)) // MB
        for vi in range(NEOT // 16):
            t_m = (ii + vi * 16) * MB
            acc = jnp.zeros((16,), jnp.int32)
            for e in range(EX):
                ge = jnp.max(jnp.where(ii == e, gend, 0))
                acc = acc + (t_m >= ge).astype(jnp.int32)
            eotv = jnp.minimum(acc, EX - 1)
            if vi * 16 <= NT < (vi + 1) * 16:
                eotv = jnp.where(ii == (NT - vi * 16), ntv, eotv)
            eot_v[pl.ds(vi * 16, 16)] = eotv
        pltpu.sync_copy(eot_v, eot_hbm)

    tok_base = jnp.where(wid >= (SL // CH), wid * CH - SL, wid * CH)
    for v in range(CH // 16):
        tok2_v[v // 2, pl.ds((v % 2) * 16, 16)] = tok_base + v * 16 + ii
    rows = [rows_v.at[0], rows_v.at[1]]
    gsem = [sem, sem2]
    ssem = [sem3, sem4]
    pltpu.async_copy(x_hbm.at[tok2_v.at[0]], rows[0], gsem[0])
    pltpu.async_copy(x_hbm.at[tok2_v.at[1]], rows[1], gsem[1])

    for v in range(CH // 16):
        ev = eid_v[pl.ds(v * 16, 16)]
        base = plsc.load_gather(cnt_v, [ev])
        rank = jnp.zeros((16,), jnp.int32)
        hist = jnp.zeros((16,), jnp.int32)
        for e in range(EX):
            m = ev == e
            mi = m.astype(jnp.int32)
            cs = plsc.cumsum(mi)
            rank = rank + jnp.where(m, cs - 1, 0)
            hist = hist + jnp.where(ii == e, jnp.sum(mi), 0)
        posv = base + rank
        pos_v[pl.ds(v * 16, 16)] = posv
        pos2_v[v // 2, pl.ds((v % 2) * 16, 16)] = posv
        cnt_v[...] = cnt_v[...] + hist
    pltpu.sync_copy(pos_v, pos_hbm.at[pl.ds(wid * CH, CH)])

    nchk = CH // 32
    for chk in range(nchk):
        b = chk % 2
        pltpu.make_async_copy(x_hbm.at[tok2_v.at[chk]], rows[b],
                              gsem[b]).wait()
        pltpu.async_copy(rows[b], xs_hbm.at[pos2_v.at[chk]], ssem[b])
        if chk + 2 < nchk:
            pltpu.make_async_copy(rows[b], xs_hbm.at[pos2_v.at[chk]],
                                  ssem[b]).wait()
            pltpu.async_copy(x_hbm.at[tok2_v.at[chk + 2]], rows[b], gsem[b])
    for chk in range(max(nchk - 2, 0), nchk):
        b = chk % 2
        pltpu.make_async_copy(rows[b], xs_hbm.at[pos2_v.at[chk]],
                              ssem[b]).wait()


def _route(eids, counts, x):
    call = functools.partial(
        pl.kernel,
        mesh=plsc.VectorSubcoreMesh(core_axis_name="c", subcore_axis_name="s"),
        out_type=[
            jax.ShapeDtypeStruct((NPAIR,), jnp.int32),
            jax.ShapeDtypeStruct((NEOT,), jnp.int32),
            jax.ShapeDtypeStruct((TBUF, DM), jnp.float32),
        ],
        scratch_types=[
            pltpu.VMEM((CH,), jnp.int32),
            pltpu.VMEM((NW, 16), jnp.int32),
            pltpu.VMEM((16,), jnp.int32),
            pltpu.VMEM((16,), jnp.int32),
            pltpu.VMEM((CH,), jnp.int32),
            pltpu.VMEM((CH // 32, 32), jnp.int32),
            pltpu.VMEM((CH // 32, 32), jnp.int32),
            pltpu.VMEM((NEOT,), jnp.int32),
            pltpu.VMEM((2, 32, DM), jnp.float32),
            pltpu.SemaphoreType.DMA,
            pltpu.SemaphoreType.DMA,
            pltpu.SemaphoreType.DMA,
            pltpu.SemaphoreType.DMA,
        ],
        compiler_params=pltpu.CompilerParams(needs_layout_passes=False),
    )
    return call(_route_body)(eids, counts, x)


def _mlp_body(eot_s, xs_ref, w1_ref, w2_ref, w3_ref, ys_ref):
    t = pl.program_id(0)
    nt = eot_s[NT]

    @pl.when(t < nt)
    def _():
        _mlp_tile(xs_ref, w1_ref, w2_ref, w3_ref, ys_ref)


def _mlp_tile(xs_ref, w1_ref, w2_ref, w3_ref, ys_ref):
    xt = xs_ref[...]
    w1 = w1_ref[0]
    w2 = w2_ref[0]
    w3 = w3_ref[0]
    h1 = lax.dot_general(xt, w1, (((1,), (1,)), ((), ())),
                         preferred_element_type=jnp.float32)
    h2 = lax.dot_general(xt, w2, (((1,), (1,)), ((), ())),
                         preferred_element_type=jnp.float32)
    h = h1 * (1.0 / (1.0 + jnp.exp(-h1))) * h2
    ys_ref[...] = lax.dot_general(h, w3, (((1,), (1,)), ((), ())),
                                  preferred_element_type=jnp.float32)


def _mlp(eot, xs, w1, w2, w3):
    grid_spec = pltpu.PrefetchScalarGridSpec(
        num_scalar_prefetch=1,
        grid=(NT,),
        in_specs=[
            pl.BlockSpec(
                (MB, DM),
                lambda t, eot_s: (jnp.minimum(t, eot_s[NT] - 1), 0)),
            pl.BlockSpec(
                (1, FF, DM),
                lambda t, eot_s: (eot_s[jnp.minimum(t, eot_s[NT] - 1)], 0, 0)),
            pl.BlockSpec(
                (1, FF, DM),
                lambda t, eot_s: (eot_s[jnp.minimum(t, eot_s[NT] - 1)], 0, 0)),
            pl.BlockSpec(
                (1, DM, FF),
                lambda t, eot_s: (eot_s[jnp.minimum(t, eot_s[NT] - 1)], 0, 0)),
        ],
        out_specs=pl.BlockSpec(
            (MB, DM), lambda t, eot_s: (jnp.minimum(t, eot_s[NT] - 1), 0)),
    )
    return pl.pallas_call(
        _mlp_body,
        grid_spec=grid_spec,
        out_shape=jax.ShapeDtypeStruct((TBUF, DM), jnp.float32),
        compiler_params=pltpu.CompilerParams(
            dimension_semantics=("arbitrary",),
            vmem_limit_bytes=100 * 1024 * 1024),
    )(eot, xs, w1, w2, w3)


def _combine_body(ys_hbm, pos_hbm, prob_hbm, out_hbm,
                  idxa_v, idxb_v, pa_v, pb_v, rowsa_v, rowsb_v, outc_v,
                  sa0, sa1, sb0, sb1, so0, so1):
    wid = lax.axis_index("s") * NC + lax.axis_index("c")
    base = wid * TPW
    pltpu.sync_copy(pos_hbm.at[pl.ds(base, TPW)], idxa_v)
    pltpu.sync_copy(pos_hbm.at[pl.ds(SL + base, TPW)], idxb_v)
    pltpu.sync_copy(prob_hbm.at[pl.ds(base, TPW)], pa_v)
    pltpu.sync_copy(prob_hbm.at[pl.ds(SL + base, TPW)], pb_v)
    nch = TPW // 16
    ra = [rowsa_v.at[0], rowsa_v.at[1]]
    rb = [rowsb_v.at[0], rowsb_v.at[1]]
    oc = [outc_v.at[0], outc_v.at[1]]
    sas = [sa0, sa1]
    sbs = [sb0, sb1]
    sos = [so0, so1]

    def ia(chk):
        return idxa_v.at[pl.ds(chk * 16, 16)]

    def ib(chk):
        return idxb_v.at[pl.ds(chk * 16, 16)]

    for chk in range(2):
        pltpu.async_copy(ys_hbm.at[ia(chk)], ra[chk], sas[chk])
        pltpu.async_copy(ys_hbm.at[ib(chk)], rb[chk], sbs[chk])
    for chk in range(nch):
        b = chk % 2
        pltpu.make_async_copy(ys_hbm.at[ia(chk)], ra[b], sas[b]).wait()
        pltpu.make_async_copy(ys_hbm.at[ib(chk)], rb[b], sbs[b]).wait()
        if chk >= 2:
            pltpu.make_async_copy(
                oc[b], out_hbm.at[pl.ds(base + (chk - 2) * 16, 16)],
                sos[b]).wait()

        def body(tt, carry):
            gt = chk * 16 + tt
            pa = plsc.load_gather(pa_v, [jnp.full((16,), gt, jnp.int32)])
            pb = plsc.load_gather(pb_v, [jnp.full((16,), gt, jnp.int32)])
            for d in range(DM // 16):
                sl = pl.ds(d * 16, 16)
                oc[b][tt, sl] = pa * ra[b][tt, sl] + pb * rb[b][tt, sl]
            return carry

        lax.fori_loop(0, 16, body, 0)
        pltpu.async_copy(oc[b], out_hbm.at[pl.ds(base + chk * 16, 16)], sos[b])
        if chk + 2 < nch:
            pltpu.async_copy(ys_hbm.at[ia(chk + 2)], ra[b], sas[b])
            pltpu.async_copy(ys_hbm.at[ib(chk + 2)], rb[b], sbs[b])
    for chk in range(nch - 2, nch):
        b = chk % 2
        pltpu.make_async_copy(
            oc[b], out_hbm.at[pl.ds(base + chk * 16, 16)], sos[b]).wait()


def _combine(ys, pos, prob):
    call = functools.partial(
        pl.kernel,
        mesh=plsc.VectorSubcoreMesh(core_axis_name="c", subcore_axis_name="s"),
        out_type=jax.ShapeDtypeStruct((SL, DM), jnp.float32),
        scratch_types=[
            pltpu.VMEM((TPW,), jnp.int32),
            pltpu.VMEM((TPW,), jnp.int32),
            pltpu.VMEM((TPW,), jnp.float32),
            pltpu.VMEM((TPW,), jnp.float32),
            pltpu.VMEM((2, 16, DM), jnp.float32),
            pltpu.VMEM((2, 16, DM), jnp.float32),
            pltpu.VMEM((2, 16, DM), jnp.float32),
            pltpu.SemaphoreType.DMA,
            pltpu.SemaphoreType.DMA,
            pltpu.SemaphoreType.DMA,
            pltpu.SemaphoreType.DMA,
            pltpu.SemaphoreType.DMA,
            pltpu.SemaphoreType.DMA,
        ],
        compiler_params=pltpu.CompilerParams(needs_layout_passes=False),
    )
    return call(_combine_body)(ys, pos, prob)


def kernel(xmat, Wg, W1, W2, W3):
    bsz, ln, _ = xmat.shape
    x = xmat.reshape(SL, DM)
    eo, po, counts = _gating(x, Wg)
    eids = eo.reshape(NPAIR)
    probs = po.reshape(NPAIR)
    pos, eot, xs = _route(eids, counts, x)
    ys = _mlp(eot, xs, W1, W2, W3)
    out = _combine(ys, pos, probs)
    return out.reshape(bsz, ln, DM)


# speedup vs baseline: 1.4973x; 1.0094x over previous
import functools
import jax
import jax.numpy as jnp
from jax import lax
from jax.experimental import pallas as pl
from jax.experimental.pallas import tpu as pltpu
from jax.experimental.pallas import tpu_sc as plsc

EX = 8
TOPK = 2
DM = 1024
FF = 2048
SL = 2048
NPAIR = SL * TOPK
MB = 256
NT = (NPAIR + EX * (MB - 1) + MB - 1) // MB
TBUF = NT * MB
NEOT = ((NT + 15) // 16) * 16

NC, NS, LN = 2, 16, 16
NW = NC * NS
CH = NPAIR // NW
TPW = SL // NW


def _iota16():
    return lax.iota(jnp.int32, 16)


def _gate_body(x_ref, wg_ref, eids_ref, probs_ref, counts_ref):
    x = x_ref[...]
    wg = wg_ref[...]
    g = lax.dot_general(x, wg, (((1,), (1,)), ((), ())),
                        preferred_element_type=jnp.float32)
    ii = lax.broadcasted_iota(jnp.int32, (SL, EX), 1)
    m0 = jnp.max(g, axis=1, keepdims=True)
    e0 = jnp.min(jnp.where(g == m0, ii, EX), axis=1, keepdims=True)
    g2 = jnp.where(ii == e0, -jnp.inf, g)
    m1 = jnp.max(g2, axis=1, keepdims=True)
    e1 = jnp.min(jnp.where(g2 == m1, ii, EX), axis=1, keepdims=True)
    b = jnp.exp(m1 - m0)
    denom = 1.0 + b
    eids_ref[pl.ds(0, SL), :] = e0
    eids_ref[pl.ds(SL, SL), :] = e1
    probs_ref[pl.ds(0, SL), :] = 1.0 / denom
    probs_ref[pl.ds(SL, SL), :] = b / denom
    i16 = lax.broadcasted_iota(jnp.int32, (SL, 16), 1)
    oh0 = (i16 == e0).astype(jnp.float32)
    oh1 = (i16 == e1).astype(jnp.float32)
    seg = lax.broadcasted_iota(jnp.int32, (SL // CH, SL), 0)
    col = lax.broadcasted_iota(jnp.int32, (SL // CH, SL), 1)
    sel = (col // CH == seg).astype(jnp.float32)
    c0 = lax.dot_general(sel, oh0, (((1,), (0,)), ((), ())),
                         preferred_element_type=jnp.float32)
    c1 = lax.dot_general(sel, oh1, (((1,), (0,)), ((), ())),
                         preferred_element_type=jnp.float32)
    counts_ref[pl.ds(0, SL // CH), :] = c0.astype(jnp.int32)
    counts_ref[pl.ds(SL // CH, SL // CH), :] = c1.astype(jnp.int32)


def _gating(x, wg):
    outs = [
        jax.ShapeDtypeStruct((NPAIR, 1), jnp.int32),
        jax.ShapeDtypeStruct((NPAIR, 1), jnp.float32),
        jax.ShapeDtypeStruct((NW, 16), jnp.int32),
    ]
    return pl.pallas_call(_gate_body, out_shape=outs)(x, wg)


def _route_body(eids_hbm, counts_hbm, x_hbm,
                pos_hbm, eot_hbm, xs_hbm,
                eid_v, call_v, cnt_v, pos_v, pos2_v, tok2_v,
                eot_v, rows_v, sem, sem2, sem3, sem4):
    wid = lax.axis_index("s") * NC + lax.axis_index("c")
    ii = _iota16()
    pltpu.sync_copy(eids_hbm.at[pl.ds(wid * CH, CH)], eid_v)
    pltpu.sync_copy(counts_hbm, call_v)

    prior = jnp.zeros((16,), jnp.int32)
    total = jnp.zeros((16,), jnp.int32)
    for w in range(NW):
        row = call_v[w]
        wv = jnp.full((16,), w, jnp.int32)
        prior = prior + jnp.where(wv < wid, row, 0)
        total = total + row
    cpad = ((total + (MB - 1)) // MB) * MB
    gend = plsc.cumsum(cpad)
    gbase = gend - cpad
    start = gbase + prior
    cnt_v[...] = start

    @pl.when(wid == 0)
    def _():
        ntv = jnp.max(jnp.where(ii == EX - 1, gend, 0)) // MB
        for vi in range(NEOT // 16):
            t_m = (ii + vi * 16) * MB
            acc = jnp.zeros((16,), jnp.int32)
            for e in range(EX):
                ge = jnp.max(jnp.where(ii == e, gend, 0))
                acc = acc + (t_m >= ge).astype(jnp.int32)
            eotv = jnp.minimum(acc, EX - 1)
            if vi * 16 <= NT < (vi + 1) * 16:
                eotv = jnp.where(ii == (NT - vi * 16), ntv, eotv)
            eot_v[pl.ds(vi * 16, 16)] = eotv
        pltpu.sync_copy(eot_v, eot_hbm)

    tok_base = jnp.where(wid >= (SL // CH), wid * CH - SL, wid * CH)
    for v in range(CH // 16):
        tok2_v[v // 2, pl.ds((v % 2) * 16, 16)] = tok_base + v * 16 + ii
    rows = [rows_v.at[0], rows_v.at[1]]
    gsem = [sem, sem2]
    ssem = [sem3, sem4]
    pltpu.async_copy(x_hbm.at[tok2_v.at[0]], rows[0], gsem[0])
    pltpu.async_copy(x_hbm.at[tok2_v.at[1]], rows[1], gsem[1])

    for v in range(CH // 16):
        ev = eid_v[pl.ds(v * 16, 16)]
        base = plsc.load_gather(cnt_v, [ev])
        rank = jnp.zeros((16,), jnp.int32)
        hist = jnp.zeros((16,), jnp.int32)
        for e in range(EX):
            m = ev == e
            mi = m.astype(jnp.int32)
            cs = plsc.cumsum(mi)
            rank = rank + jnp.where(m, cs - 1, 0)
            hist = hist + jnp.where(ii == e, jnp.sum(mi), 0)
        posv = base + rank
        pos_v[pl.ds(v * 16, 16)] = posv
        pos2_v[v // 2, pl.ds((v % 2) * 16, 16)] = posv
        cnt_v[...] = cnt_v[...] + hist
    pltpu.sync_copy(pos_v, pos_hbm.at[pl.ds(wid * CH, CH)])

    nchk = CH // 32
    for chk in range(nchk):
        b = chk % 2
        pltpu.make_async_copy(x_hbm.at[tok2_v.at[chk]], rows[b],
                              gsem[b]).wait()
        pltpu.async_copy(rows[b], xs_hbm.at[pos2_v.at[chk]], ssem[b])
        if chk + 2 < nchk:
            pltpu.make_async_copy(rows[b], xs_hbm.at[pos2_v.at[chk]],
                                  ssem[b]).wait()
            pltpu.async_copy(x_hbm.at[tok2_v.at[chk + 2]], rows[b], gsem[b])
    for chk in range(max(nchk - 2, 0), nchk):
        b = chk % 2
        pltpu.make_async_copy(rows[b], xs_hbm.at[pos2_v.at[chk]],
                              ssem[b]).wait()


def _route(eids, counts, x):
    call = functools.partial(
        pl.kernel,
        mesh=plsc.VectorSubcoreMesh(core_axis_name="c", subcore_axis_name="s"),
        out_type=[
            jax.ShapeDtypeStruct((NPAIR,), jnp.int32),
            jax.ShapeDtypeStruct((NEOT,), jnp.int32),
            jax.ShapeDtypeStruct((TBUF, DM), jnp.float32),
        ],
        scratch_types=[
            pltpu.VMEM((CH,), jnp.int32),
            pltpu.VMEM((NW, 16), jnp.int32),
            pltpu.VMEM((16,), jnp.int32),
            pltpu.VMEM((CH,), jnp.int32),
            pltpu.VMEM((CH // 32, 32), jnp.int32),
            pltpu.VMEM((CH // 32, 32), jnp.int32),
            pltpu.VMEM((NEOT,), jnp.int32),
            pltpu.VMEM((2, 32, DM), jnp.float32),
            pltpu.SemaphoreType.DMA,
            pltpu.SemaphoreType.DMA,
            pltpu.SemaphoreType.DMA,
            pltpu.SemaphoreType.DMA,
        ],
        compiler_params=pltpu.CompilerParams(needs_layout_passes=False),
    )
    return call(_route_body)(eids, counts, x)


def _mlp_body(eot_s, xs_ref, w1_ref, w2_ref, w3_ref, ys_ref):
    t = pl.program_id(0)
    nt = eot_s[NT]

    @pl.when(t < nt)
    def _():
        _mlp_tile(xs_ref, w1_ref, w2_ref, w3_ref, ys_ref)


def _mlp_tile(xs_ref, w1_ref, w2_ref, w3_ref, ys_ref):
    xt = xs_ref[...]
    w1 = w1_ref[0]
    w2 = w2_ref[0]
    w3 = w3_ref[0]
    h1 = lax.dot_general(xt, w1, (((1,), (1,)), ((), ())),
                         preferred_element_type=jnp.float32)
    h2 = lax.dot_general(xt, w2, (((1,), (1,)), ((), ())),
                         preferred_element_type=jnp.float32)
    h = h1 * (1.0 / (1.0 + jnp.exp(-h1))) * h2
    ys_ref[...] = lax.dot_general(h, w3, (((1,), (1,)), ((), ())),
                                  preferred_element_type=jnp.float32)


def _mlp(eot, xs, w1, w2, w3):
    grid_spec = pltpu.PrefetchScalarGridSpec(
        num_scalar_prefetch=1,
        grid=(NT,),
        in_specs=[
            pl.BlockSpec(
                (MB, DM),
                lambda t, eot_s: (jnp.minimum(t, eot_s[NT] - 1), 0)),
            pl.BlockSpec(
                (1, FF, DM),
                lambda t, eot_s: (eot_s[jnp.minimum(t, eot_s[NT] - 1)], 0, 0)),
            pl.BlockSpec(
                (1, FF, DM),
                lambda t, eot_s: (eot_s[jnp.minimum(t, eot_s[NT] - 1)], 0, 0)),
            pl.BlockSpec(
                (1, DM, FF),
                lambda t, eot_s: (eot_s[jnp.minimum(t, eot_s[NT] - 1)], 0, 0)),
        ],
        out_specs=pl.BlockSpec(
            (MB, DM), lambda t, eot_s: (jnp.minimum(t, eot_s[NT] - 1), 0)),
    )
    return pl.pallas_call(
        _mlp_body,
        grid_spec=grid_spec,
        out_shape=jax.ShapeDtypeStruct((TBUF, DM), jnp.float32),
        compiler_params=pltpu.CompilerParams(
            dimension_semantics=("arbitrary",),
            vmem_limit_bytes=100 * 1024 * 1024),
    )(eot, xs, w1, w2, w3)


def _combine_body(ys_hbm, pos_hbm, prob_hbm, out_hbm,
                  idxa_v, idxb_v, pa_v, pb_v, rowsa_v, rowsb_v, outc_v,
                  sa0, sa1, sb0, sb1, so0, so1):
    wid = lax.axis_index("s") * NC + lax.axis_index("c")
    base = wid * TPW
    pltpu.sync_copy(pos_hbm.at[pl.ds(base, TPW)], idxa_v)
    pltpu.sync_copy(pos_hbm.at[pl.ds(SL + base, TPW)], idxb_v)
    pltpu.sync_copy(prob_hbm.at[pl.ds(base, TPW)], pa_v)
    pltpu.sync_copy(prob_hbm.at[pl.ds(SL + base, TPW)], pb_v)
    nch = TPW // 16
    ra = [rowsa_v.at[0], rowsa_v.at[1]]
    rb = [rowsb_v.at[0], rowsb_v.at[1]]
    oc = [outc_v.at[0], outc_v.at[1]]
    sas = [sa0, sa1]
    sbs = [sb0, sb1]
    sos = [so0, so1]

    def ia(chk):
        return idxa_v.at[pl.ds(chk * 16, 16)]

    def ib(chk):
        return idxb_v.at[pl.ds(chk * 16, 16)]

    for chk in range(2):
        pltpu.async_copy(ys_hbm.at[ia(chk)], ra[chk], sas[chk])
        pltpu.async_copy(ys_hbm.at[ib(chk)], rb[chk], sbs[chk])
    for chk in range(nch):
        b = chk % 2
        pltpu.make_async_copy(ys_hbm.at[ia(chk)], ra[b], sas[b]).wait()
        pltpu.make_async_copy(ys_hbm.at[ib(chk)], rb[b], sbs[b]).wait()
        if chk >= 2:
            pltpu.make_async_copy(
                oc[b], out_hbm.at[pl.ds(base + (chk - 2) * 16, 16)],
                sos[b]).wait()

        def body(tt, carry):
            gt = chk * 16 + tt
            pa = plsc.load_gather(pa_v, [jnp.full((16,), gt, jnp.int32)])
            pb = plsc.load_gather(pb_v, [jnp.full((16,), gt, jnp.int32)])
            for d in range(DM // 16):
                sl = pl.ds(d * 16, 16)
                oc[b][tt, sl] = pa * ra[b][tt, sl] + pb * rb[b][tt, sl]
            return carry

        lax.fori_loop(0, 16, body, 0)
        pltpu.async_copy(oc[b], out_hbm.at[pl.ds(base + chk * 16, 16)], sos[b])
        if chk + 2 < nch:
            pltpu.async_copy(ys_hbm.at[ia(chk + 2)], ra[b], sas[b])
            pltpu.async_copy(ys_hbm.at[ib(chk + 2)], rb[b], sbs[b])
    for chk in range(nch - 2, nch):
        b = chk % 2
        pltpu.make_async_copy(
            oc[b], out_hbm.at[pl.ds(base + chk * 16, 16)], sos[b]).wait()


def _combine(ys, pos, prob):
    call = functools.partial(
        pl.kernel,
        mesh=plsc.VectorSubcoreMesh(core_axis_name="c", subcore_axis_name="s"),
        out_type=jax.ShapeDtypeStruct((SL, DM), jnp.float32),
        scratch_types=[
            pltpu.VMEM((TPW,), jnp.int32),
            pltpu.VMEM((TPW,), jnp.int32),
            pltpu.VMEM((TPW,), jnp.float32),
            pltpu.VMEM((TPW,), jnp.float32),
            pltpu.VMEM((2, 16, DM), jnp.float32),
            pltpu.VMEM((2, 16, DM), jnp.float32),
            pltpu.VMEM((2, 16, DM), jnp.float32),
            pltpu.SemaphoreType.DMA,
            pltpu.SemaphoreType.DMA,
            pltpu.SemaphoreType.DMA,
            pltpu.SemaphoreType.DMA,
            pltpu.SemaphoreType.DMA,
            pltpu.SemaphoreType.DMA,
        ],
        compiler_params=pltpu.CompilerParams(needs_layout_passes=False),
    )
    return call(_combine_body)(ys, pos, prob)


def kernel(xmat, Wg, W1, W2, W3):
    bsz, ln, _ = xmat.shape
    x = xmat.reshape(SL, DM)
    eo, po, counts = _gating(x, Wg)
    eids = eo.reshape(NPAIR)
    probs = po.reshape(NPAIR)
    pos, eot, xs = _route(eids, counts, x)
    ys = _mlp(eot, xs, W1, W2, W3)
    out = _combine(ys, pos, probs)
    return out.reshape(bsz, ln, DM)
